# baseline (speedup 1.0000x reference)
import jax
import jax.numpy as jnp
from jax import lax
from jax.experimental import pallas as pl
from jax.experimental.pallas import tpu as pltpu

DIM = 64
INPUT_DIM = 128
BCOL = 128


def _body(tid_ref, tt_ref, x_ref, w_ref, o_ref):
    lane = tid_ref[0] % BCOL
    mask = lax.broadcasted_iota(jnp.int32, (DIM, BCOL), 1) == lane
    base = jnp.sum(
        jnp.where(mask, tt_ref[...], 0.0), axis=1, keepdims=True
    )
    proj = jnp.sum(w_ref[...] * x_ref[...], axis=1, keepdims=True)
    v = base + proj
    ssq = jnp.sum(v * v)
    r = jnp.minimum(lax.rsqrt(ssq), jnp.float32(1e12))
    o_ref[...] = v * r


@jax.jit
def _run(tid, table_t, x, W):
    grid_spec = pltpu.PrefetchScalarGridSpec(
        num_scalar_prefetch=1,
        grid=(1,),
        in_specs=[
            pl.BlockSpec((DIM, BCOL), lambda i, tid_ref: (0, tid_ref[0] // BCOL)),
            pl.BlockSpec((1, INPUT_DIM), lambda i, tid_ref: (0, 0)),
            pl.BlockSpec((DIM, INPUT_DIM), lambda i, tid_ref: (0, 0)),
        ],
        out_specs=pl.BlockSpec((DIM, 1), lambda i, tid_ref: (0, 0)),
    )
    out = pl.pallas_call(
        _body,
        grid_spec=grid_spec,
        out_shape=jax.ShapeDtypeStruct((DIM, 1), jnp.float32),
    )(tid, table_t, x, W)
    return out.reshape(DIM)


def kernel(prompt, task_id, input_features, table, W):
    tid = jnp.asarray(task_id, jnp.int32).reshape(1)
    x = input_features.astype(jnp.float32).reshape(1, INPUT_DIM)
    return _run(tid, table.T, x, W.astype(jnp.float32))


# speedup vs baseline: 1.0056x; 1.0056x over previous
import jax
import jax.numpy as jnp
from jax import lax
from jax.experimental import pallas as pl
from jax.experimental.pallas import tpu as pltpu

DIM = 64
INPUT_DIM = 128
BCOL = 128


def _body(tid_ref, tt_ref, x_ref, w_ref, o_ref):
    lane = tid_ref[0] % BCOL
    mask = lax.broadcasted_iota(jnp.int32, (DIM, BCOL), 1) == lane
    v = jnp.sum(
        jnp.where(mask, tt_ref[...], 0.0) + w_ref[...] * x_ref[...],
        axis=1,
        keepdims=True,
    )
    ssq = jnp.sum(v * v)
    r = jnp.minimum(lax.rsqrt(ssq), jnp.float32(1e12))
    o_ref[...] = v * r


@jax.jit
def _run(tid, table_t, x, W):
    grid_spec = pltpu.PrefetchScalarGridSpec(
        num_scalar_prefetch=1,
        grid=(1,),
        in_specs=[
            pl.BlockSpec((DIM, BCOL), lambda i, tid_ref: (0, tid_ref[0] // BCOL)),
            pl.BlockSpec((1, INPUT_DIM), lambda i, tid_ref: (0, 0)),
            pl.BlockSpec((DIM, INPUT_DIM), lambda i, tid_ref: (0, 0)),
        ],
        out_specs=pl.BlockSpec((DIM, 1), lambda i, tid_ref: (0, 0)),
    )
    out = pl.pallas_call(
        _body,
        grid_spec=grid_spec,
        out_shape=jax.ShapeDtypeStruct((DIM, 1), jnp.float32),
    )(tid, table_t, x, W)
    return out.reshape(DIM)


def kernel(prompt, task_id, input_features, table, W):
    tid = jnp.asarray(task_id, jnp.int32).reshape(1)
    x = input_features.astype(jnp.float32).reshape(1, INPUT_DIM)
    return _run(tid, table.T, x, W.astype(jnp.float32))


# speedup vs baseline: 1.6587x; 1.6495x over previous
import jax
import jax.numpy as jnp
from jax import lax
from jax.experimental import pallas as pl
from jax.experimental.pallas import tpu as pltpu

DIM = 64
INPUT_DIM = 128
BCOL = 128


def _body(tid_ref, tt_ref, x_ref, w_ref, o_ref):
    lane = tid_ref[0] % BCOL
    onehot = (
        lax.broadcasted_iota(jnp.int32, (1, BCOL), 1) == lane
    ).astype(jnp.float32)
    dn = (((1,), (1,)), ((), ()))
    base = lax.dot_general(
        onehot, tt_ref[...], dn, preferred_element_type=jnp.float32
    )
    proj = lax.dot_general(
        x_ref[...], w_ref[...], dn, preferred_element_type=jnp.float32
    )
    v = (base + proj)[0]
    ssq = jnp.sum(v * v)
    r = jnp.minimum(lax.rsqrt(ssq), jnp.float32(1e12))
    o_ref[...] = v * r


@jax.jit
def _run(tid, table_t, x, W):
    grid_spec = pltpu.PrefetchScalarGridSpec(
        num_scalar_prefetch=1,
        grid=(1,),
        in_specs=[
            pl.BlockSpec((DIM, BCOL), lambda i, tid_ref: (0, tid_ref[0] // BCOL)),
            pl.BlockSpec((1, INPUT_DIM), lambda i, tid_ref: (0, 0)),
            pl.BlockSpec((DIM, INPUT_DIM), lambda i, tid_ref: (0, 0)),
        ],
        out_specs=pl.BlockSpec((DIM,), lambda i, tid_ref: (0,)),
    )
    return pl.pallas_call(
        _body,
        grid_spec=grid_spec,
        out_shape=jax.ShapeDtypeStruct((DIM,), jnp.float32),
    )(tid, table_t, x, W)


def kernel(prompt, task_id, input_features, table, W):
    tid = jnp.asarray(task_id, jnp.int32).reshape(1)
    x = input_features.astype(jnp.float32).reshape(1, INPUT_DIM)
    return _run(tid, table.T, x, W.astype(jnp.float32))


# speedup vs baseline: 1.6592x; 1.0003x over previous
import functools

import jax
import jax.numpy as jnp
from jax import lax
from jax.experimental import pallas as pl
from jax.experimental.pallas import tpu as pltpu

DIM = 64
INPUT_DIM = 128
BCOL = 128


def _body(tid_ref, tt_ref, x_ref, w_ref, o_ref, *, ncols):
    lane = tid_ref[0] % BCOL
    colid = lax.broadcasted_iota(jnp.int32, (1, BCOL), 1)
    onehot = (colid == lane).astype(jnp.float32)
    valid = colid < ncols - (tid_ref[0] // BCOL) * BCOL
    tile = jnp.where(valid, tt_ref[...], 0.0)
    dn = (((1,), (1,)), ((), ()))
    base = lax.dot_general(
        onehot, tile, dn, preferred_element_type=jnp.float32
    )
    proj = lax.dot_general(
        x_ref[...], w_ref[...], dn, preferred_element_type=jnp.float32
    )
    v = (base + proj)[0]
    ssq = jnp.sum(v * v)
    r = jnp.minimum(lax.rsqrt(ssq), jnp.float32(1e12))
    o_ref[...] = v * r


@jax.jit
def _run(tid, table_t, x, W):
    grid_spec = pltpu.PrefetchScalarGridSpec(
        num_scalar_prefetch=1,
        grid=(1,),
        in_specs=[
            pl.BlockSpec((DIM, BCOL), lambda i, tid_ref: (0, tid_ref[0] // BCOL)),
            pl.BlockSpec((1, INPUT_DIM), lambda i, tid_ref: (0, 0)),
            pl.BlockSpec((DIM, INPUT_DIM), lambda i, tid_ref: (0, 0)),
        ],
        out_specs=pl.BlockSpec((DIM,), lambda i, tid_ref: (0,)),
    )
    body = functools.partial(_body, ncols=table_t.shape[1])
    return pl.pallas_call(
        body,
        grid_spec=grid_spec,
        out_shape=jax.ShapeDtypeStruct((DIM,), jnp.float32),
    )(tid, table_t, x, W)


def kernel(prompt, task_id, input_features, table, W):
    tid = jnp.asarray(task_id, jnp.int32).reshape(1)
    x = input_features.astype(jnp.float32).reshape(1, INPUT_DIM)
    return _run(tid, table.T, x, W.astype(jnp.float32))
